# baseline (speedup 1.0000x reference)
import jax
import jax.numpy as jnp
from jax import lax
from jax.experimental import pallas as pl
from jax.experimental.pallas import tpu as pltpu
from jax.experimental.pallas import tpu_sc as plsc

_N_NODES = 10000
_MAX_DEG = 32
_BATCH = 4096
_K = 16

_NC, _NS, _L = 2, 16, 16
_NW = _NC * _NS
_BPW = _BATCH // _NW
_EPW = _BPW * _MAX_DEG
_NE = _BATCH * _MAX_DEG
_GE = 128
_NGRP = _EPW // _GE

_DN = lax.GatherDimensionNumbers(
    offset_dims=(), collapsed_slice_dims=(0,), start_index_map=(0,))


def _splat(vec, lane):
    idx = jnp.full((_L, 1), lane, jnp.int32)
    return lax.gather(vec, idx, _DN, slice_sizes=(1,),
                      mode=lax.GatherScatterMode.PROMISE_IN_BOUNDS)


def _sc1_body(ids_hbm, adj_hbm, adj_out, ids_v, adj_v, flat_v, sem):
    wid = lax.axis_index("s") * _NC + lax.axis_index("c")
    base = wid * _BPW

    pltpu.sync_copy(ids_hbm.at[pl.ds(base, _BPW)], ids_v)
    pltpu.async_copy(adj_hbm.at[ids_v], adj_v, sem).wait()

    def flatten(i, carry):
        c0 = adj_v[i, pl.ds(0, _L)]
        c1 = adj_v[i, pl.ds(_L, _L)]
        flat_v[pl.ds(i * _MAX_DEG, _L)] = c0
        flat_v[pl.ds(i * _MAX_DEG + _L, _L)] = c1
        return carry

    lax.fori_loop(0, _BPW, flatten, 0)
    pltpu.sync_copy(flat_v, adj_out.at[pl.ds(base * _MAX_DEG, _EPW)])


def _sc_adj(ids, adj_info):
    kern = pl.kernel(
        _sc1_body,
        out_type=jax.ShapeDtypeStruct((_NE,), jnp.int32),
        mesh=plsc.VectorSubcoreMesh(core_axis_name="c", subcore_axis_name="s"),
        compiler_params=pltpu.CompilerParams(use_tc_tiling_on_sc=False),
        scratch_types=[
            pltpu.VMEM((_BPW,), jnp.int32),
            pltpu.VMEM((_BPW, _MAX_DEG), jnp.int32),
            pltpu.VMEM((_EPW,), jnp.int32),
            pltpu.SemaphoreType.DMA,
        ],
    )
    return kern(ids, adj_info)


def _fire(g, prob_hbm, rows_v, adjv, buf, sem):
    for k in range(_GE // _L):
        base = g * _GE + k * _L
        vr = rows_v[pl.ds(base, _L)]
        vc = adjv[pl.ds(base, _L)]
        for sj in range(_L):
            r = vr[sj]
            c = vc[sj]
            cb = pl.multiple_of((c // 128) * 128, 128)
            pltpu.async_copy(prob_hbm.at[r, pl.ds(cb, 128)],
                             buf.at[k * _L + sj], sem)


def _extract(g, adjv, buf, sel_v):
    lanes = lax.broadcasted_iota(jnp.int32, (_L,), 0)
    for k in range(_GE // _L):
        base = g * _GE + k * _L
        vc = adjv[pl.ds(base, _L)]
        acc = jnp.zeros((_L,), jnp.float32)
        for sj in range(_L):
            cm = vc[sj] % 128
            part = buf[k * _L + sj, pl.ds((cm // _L) * _L, _L)]
            val = _splat(part, cm % _L)
            acc = jnp.where(lanes == sj, val, acc)
        sel_v[pl.ds(base, _L)] = acc


def _sc2_body(ids_hbm, adjf_hbm, prob_hbm, sel_out,
              ids_v, rows_v, adjv, sel_v, buf0, buf1, sem0, sem1):
    wid = lax.axis_index("s") * _NC + lax.axis_index("c")
    base = wid * _BPW

    pltpu.sync_copy(ids_hbm.at[pl.ds(base, _BPW)], ids_v)
    pltpu.sync_copy(adjf_hbm.at[pl.ds(base * _MAX_DEG, _EPW)], adjv)

    def rowsplat(i, carry):
        vec = ids_v[pl.ds((i // _L) * _L, _L)]
        spl = _splat(vec, i % _L)
        rows_v[pl.ds(i * _MAX_DEG, _L)] = spl
        rows_v[pl.ds(i * _MAX_DEG + _L, _L)] = spl
        return carry

    lax.fori_loop(0, _BPW, rowsplat, 0)

    dummy = prob_hbm.at[pl.ds(0, _GE), pl.ds(0, 128)]
    _fire(0, prob_hbm, rows_v, adjv, buf0, sem0)

    def pipe(g2, carry):
        a = 2 * g2
        _fire(a + 1, prob_hbm, rows_v, adjv, buf1, sem1)
        pltpu.make_async_copy(dummy, buf0, sem0).wait()
        _extract(a, adjv, buf0, sel_v)

        @pl.when(g2 < _NGRP // 2 - 1)
        def _():
            _fire(a + 2, prob_hbm, rows_v, adjv, buf0, sem0)

        pltpu.make_async_copy(dummy, buf1, sem1).wait()
        _extract(a + 1, adjv, buf1, sel_v)
        return carry

    lax.fori_loop(0, _NGRP // 2, pipe, 0)

    pltpu.sync_copy(sel_v, sel_out.at[pl.ds(base * _MAX_DEG, _EPW)])


def _sc_probs(ids, adj_flat, prob_matrix):
    kern = pl.kernel(
        _sc2_body,
        out_type=jax.ShapeDtypeStruct((_NE,), jnp.float32),
        mesh=plsc.VectorSubcoreMesh(core_axis_name="c", subcore_axis_name="s"),
        scratch_types=[
            pltpu.VMEM((_BPW,), jnp.int32),
            pltpu.VMEM((_EPW,), jnp.int32),
            pltpu.VMEM((_EPW,), jnp.int32),
            pltpu.VMEM((_EPW,), jnp.float32),
            pltpu.VMEM((_GE, 128), jnp.float32),
            pltpu.VMEM((_GE, 128), jnp.float32),
            pltpu.SemaphoreType.DMA,
            pltpu.SemaphoreType.DMA,
        ],
    )
    return kern(ids, adj_flat, prob_matrix)


def _tc_body(selT_ref, adjT_ref, out_ref):
    sel = selT_ref[...]
    adj = adjT_ref[...]
    jio = lax.broadcasted_iota(jnp.int32, (_MAX_DEG, _BATCH), 0)
    rank = jnp.zeros((_MAX_DEG, _BATCH), jnp.int32)
    for k in range(_MAX_DEG):
        ck = sel[k:k + 1, :]
        gt = (ck > sel).astype(jnp.int32)
        eq = jnp.logical_and(ck == sel, k < jio).astype(jnp.int32)
        rank = rank + gt + eq
    rows = []
    for p in range(_K):
        rows.append(jnp.sum(jnp.where(rank == p, adj, 0), axis=0,
                            keepdims=True))
    out_ref[...] = jnp.concatenate(rows, axis=0)


def _tc_topk(selT, adjT):
    return pl.pallas_call(
        _tc_body,
        out_shape=jax.ShapeDtypeStruct((_K, _BATCH), jnp.int32),
    )(selT, adjT)


def kernel(ids, num_samples, num, adj_info, prob_matrix):
    adj_flat = _sc_adj(ids, adj_info)
    sel_flat = _sc_probs(ids, adj_flat, prob_matrix)
    selT = sel_flat.reshape(_BATCH, _MAX_DEG).T
    adjT = adj_flat.reshape(_BATCH, _MAX_DEG).T
    outT = _tc_topk(selT, adjT)
    sample_val = outT.T
    return sample_val + jnp.asarray(num_samples - _K, dtype=sample_val.dtype)
